# Initial kernel scaffold; baseline (speedup 1.0000x reference)
import functools

import jax
import jax.numpy as jnp
from jax import lax
from jax.experimental import pallas as pl
from jax.experimental.pallas import tpu as pltpu
from jax.experimental.pallas import tpu_sc as plsc

VOCAB = 21128
DIM = 768
BATCH = 4096
SEQ = 50
B = BATCH * SEQ

_info = plsc.get_sparse_core_info()
NC, NS = _info.num_cores, _info.num_subcores
NW = NC * NS
B_PER_W = B // NW
CHUNK = 64
N_CHUNKS = B_PER_W // CHUNK


def _make_kernel():
    mesh = plsc.VectorSubcoreMesh(core_axis_name="c", subcore_axis_name="s")

    @functools.partial(
        pl.kernel,
        out_type=jax.ShapeDtypeStruct((B, DIM), jnp.float32),
        mesh=mesh,
        scratch_types=[
            pltpu.VMEM((B_PER_W,), jnp.int32),
            pltpu.VMEM((2, CHUNK, DIM), jnp.float32),
            pltpu.SemaphoreType.DMA,
            pltpu.SemaphoreType.DMA,
        ],
    )
    def k(idx_hbm, table_hbm, out_hbm, idx_v, rows_v, gsem, ssem):
        wid = lax.axis_index("s") * NC + lax.axis_index("c")
        base = wid * B_PER_W
        pltpu.sync_copy(idx_hbm.at[pl.ds(base, B_PER_W)], idx_v)

        pltpu.async_copy(
            table_hbm.at[idx_v.at[pl.ds(0, CHUNK)]], rows_v.at[0], gsem
        )

        def body(j, _):
            slot = lax.rem(j, 2)
            nslot = lax.rem(j + 1, 2)

            @pl.when(j + 1 < N_CHUNKS)
            def _():
                pltpu.async_copy(
                    table_hbm.at[idx_v.at[pl.ds((j + 1) * CHUNK, CHUNK)]],
                    rows_v.at[nslot],
                    gsem,
                )

            pltpu.make_async_copy(
                table_hbm.at[idx_v.at[pl.ds(j * CHUNK, CHUNK)]],
                rows_v.at[slot],
                gsem,
            ).wait()

            @pl.when(j >= 2)
            def _():
                pltpu.make_async_copy(
                    rows_v.at[slot],
                    out_hbm.at[pl.ds(base + (j - 2) * CHUNK, CHUNK)],
                    ssem,
                ).wait()

            pltpu.async_copy(
                rows_v.at[slot],
                out_hbm.at[pl.ds(base + j * CHUNK, CHUNK)],
                ssem,
            )
            return 0

        lax.fori_loop(0, N_CHUNKS, body, 0, unroll=False)
        pltpu.make_async_copy(
            rows_v.at[lax.rem(N_CHUNKS - 2, 2)],
            out_hbm.at[pl.ds(base + (N_CHUNKS - 2) * CHUNK, CHUNK)],
            ssem,
        ).wait()
        pltpu.make_async_copy(
            rows_v.at[lax.rem(N_CHUNKS - 1, 2)],
            out_hbm.at[pl.ds(base + (N_CHUNKS - 1) * CHUNK, CHUNK)],
            ssem,
        ).wait()

    return k


_gather = _make_kernel()


@jax.jit
def kernel(token_ids, table):
    flat_ids = token_ids.reshape(B).astype(jnp.int32)
    out = _gather(flat_ids, table)
    return out.reshape(BATCH, SEQ, DIM)


# speedup vs baseline: 1.3275x; 1.3275x over previous
import functools

import jax
import jax.numpy as jnp
from jax import lax
from jax.experimental import pallas as pl
from jax.experimental.pallas import tpu as pltpu
from jax.experimental.pallas import tpu_sc as plsc

VOCAB = 21128
DIM = 768
BATCH = 4096
SEQ = 50
B = BATCH * SEQ

_info = plsc.get_sparse_core_info()
NC, NS = _info.num_cores, _info.num_subcores
NW = NC * NS
B_PER_W = B // NW
CHUNK = 64
N_CHUNKS = B_PER_W // CHUNK


def _make_kernel():
    mesh = plsc.VectorSubcoreMesh(core_axis_name="c", subcore_axis_name="s")

    @functools.partial(
        pl.kernel,
        out_type=jax.ShapeDtypeStruct((B, DIM), jnp.float32),
        mesh=mesh,
        scratch_types=[
            pltpu.VMEM((B_PER_W,), jnp.int32),
            pltpu.VMEM((2, CHUNK, DIM), jnp.float32),
            pltpu.SemaphoreType.DMA,
            pltpu.SemaphoreType.DMA,
            pltpu.SemaphoreType.DMA,
            pltpu.SemaphoreType.DMA,
        ],
    )
    def k(idx_hbm, table_hbm, out_hbm, idx_v, rows_v, g0, g1, s0, s1):
        wid = lax.axis_index("s") * NC + lax.axis_index("c")
        base = wid * B_PER_W
        pltpu.sync_copy(idx_hbm.at[pl.ds(base, B_PER_W)], idx_v)

        def gather(j, slot, sem):
            pltpu.async_copy(
                table_hbm.at[idx_v.at[pl.ds(j * CHUNK, CHUNK)]],
                rows_v.at[slot],
                sem,
            )

        def wait_gather(j, slot, sem):
            pltpu.make_async_copy(
                table_hbm.at[idx_v.at[pl.ds(j * CHUNK, CHUNK)]],
                rows_v.at[slot],
                sem,
            ).wait()

        def store(j, slot, sem):
            pltpu.async_copy(
                rows_v.at[slot],
                out_hbm.at[pl.ds(base + j * CHUNK, CHUNK)],
                sem,
            )

        def wait_store(j, slot, sem):
            pltpu.make_async_copy(
                rows_v.at[slot],
                out_hbm.at[pl.ds(base + j * CHUNK, CHUNK)],
                sem,
            ).wait()

        gather(0, 0, g0)

        def body(jj, _):
            j0 = 2 * jj
            j1 = j0 + 1

            @pl.when(jj >= 1)
            def _():
                wait_store(j0 - 1, 1, s1)

            gather(j1, 1, g1)
            wait_gather(j0, 0, g0)
            store(j0, 0, s0)

            @pl.when(jj < N_CHUNKS // 2 - 1)
            def _():
                wait_store(j0, 0, s0)
                gather(j1 + 1, 0, g0)

            wait_gather(j1, 1, g1)
            store(j1, 1, s1)
            return 0

        lax.fori_loop(0, N_CHUNKS // 2, body, 0, unroll=False)
        wait_store(N_CHUNKS - 2, 0, s0)
        wait_store(N_CHUNKS - 1, 1, s1)

    return k


_gather_fn = _make_kernel()


@jax.jit
def kernel(token_ids, table):
    flat_ids = token_ids.reshape(B).astype(jnp.int32)
    out = _gather_fn(flat_ids, table)
    return out.reshape(BATCH, SEQ, DIM)


# speedup vs baseline: 1.3285x; 1.0008x over previous
import functools

import jax
import jax.numpy as jnp
from jax import lax
from jax.experimental import pallas as pl
from jax.experimental.pallas import tpu as pltpu
from jax.experimental.pallas import tpu_sc as plsc

VOCAB = 21128
DIM = 768
BATCH = 4096
SEQ = 50
B = BATCH * SEQ

_info = plsc.get_sparse_core_info()
NC, NS = _info.num_cores, _info.num_subcores
NW = NC * NS
B_PER_W = B // NW
CHUNK = 32
N_CHUNKS = B_PER_W // CHUNK
NBUF = 4
LEAD = 2
N_GROUPS = N_CHUNKS // NBUF


def _make_kernel():
    mesh = plsc.VectorSubcoreMesh(core_axis_name="c", subcore_axis_name="s")

    @functools.partial(
        pl.kernel,
        out_type=jax.ShapeDtypeStruct((B, DIM), jnp.float32),
        mesh=mesh,
        scratch_types=[
            pltpu.VMEM((B_PER_W,), jnp.int32),
            pltpu.VMEM((NBUF, CHUNK, DIM), jnp.float32),
            [pltpu.SemaphoreType.DMA] * NBUF,
            [pltpu.SemaphoreType.DMA] * NBUF,
        ],
    )
    def k(idx_hbm, table_hbm, out_hbm, idx_v, rows_v, gsems, ssems):
        wid = lax.axis_index("s") * NC + lax.axis_index("c")
        base = wid * B_PER_W
        pltpu.sync_copy(idx_hbm.at[pl.ds(base, B_PER_W)], idx_v)

        def gather(j, slot):
            pltpu.async_copy(
                table_hbm.at[idx_v.at[pl.ds(j * CHUNK, CHUNK)]],
                rows_v.at[slot],
                gsems[slot],
            )

        def wait_gather(j, slot):
            pltpu.make_async_copy(
                table_hbm.at[idx_v.at[pl.ds(j * CHUNK, CHUNK)]],
                rows_v.at[slot],
                gsems[slot],
            ).wait()

        def store(j, slot):
            pltpu.async_copy(
                rows_v.at[slot],
                out_hbm.at[pl.ds(base + j * CHUNK, CHUNK)],
                ssems[slot],
            )

        def wait_store(j, slot):
            pltpu.make_async_copy(
                rows_v.at[slot],
                out_hbm.at[pl.ds(base + j * CHUNK, CHUNK)],
                ssems[slot],
            ).wait()

        for b in range(LEAD):
            gather(b, b)

        def body(jj, _):
            for b in range(NBUF):
                j = jj * NBUF + b
                ns = (b + LEAD) % NBUF

                @pl.when(j + LEAD - NBUF >= 0)
                def _():
                    wait_store(j + LEAD - NBUF, ns)

                @pl.when(j + LEAD < N_CHUNKS)
                def _():
                    gather(j + LEAD, ns)

                wait_gather(j, b)
                store(j, b)
            return 0

        lax.fori_loop(0, N_GROUPS, body, 0, unroll=False)
        for t in range(NBUF - LEAD):
            j = N_CHUNKS - (NBUF - LEAD) + t
            wait_store(j, j % NBUF)

    return k


_gather_fn = _make_kernel()


@jax.jit
def kernel(token_ids, table):
    flat_ids = token_ids.reshape(B).astype(jnp.int32)
    out = _gather_fn(flat_ids, table)
    return out.reshape(BATCH, SEQ, DIM)


# speedup vs baseline: 2.1559x; 1.6228x over previous
import functools

import jax
import jax.numpy as jnp
from jax import lax
from jax.experimental import pallas as pl
from jax.experimental.pallas import tpu as pltpu
from jax.experimental.pallas import tpu_sc as plsc

VOCAB = 21128
DIM = 768
BATCH = 4096
SEQ = 50

_info = plsc.get_sparse_core_info()
NC, NS = _info.num_cores, _info.num_subcores
NW = NC * NS
ROWS_PER_W = BATCH // NW
NBUF = 2


def _make_kernel():
    mesh = plsc.VectorSubcoreMesh(core_axis_name="c", subcore_axis_name="s")

    @functools.partial(
        pl.kernel,
        out_type=jax.ShapeDtypeStruct((BATCH, SEQ, DIM), jnp.float32),
        mesh=mesh,
        scratch_types=[
            pltpu.VMEM((ROWS_PER_W, SEQ), jnp.int32),
            pltpu.VMEM((NBUF, SEQ, DIM), jnp.float32),
            [pltpu.SemaphoreType.DMA] * NBUF,
            [pltpu.SemaphoreType.DMA] * NBUF,
        ],
    )
    def k(idx_hbm, table_hbm, out_hbm, idx_v, rows_v, gsems, ssems):
        wid = lax.axis_index("s") * NC + lax.axis_index("c")
        base = wid * ROWS_PER_W
        pltpu.sync_copy(idx_hbm.at[pl.ds(base, ROWS_PER_W)], idx_v)

        def gather(j, slot):
            pltpu.async_copy(
                table_hbm.at[idx_v.at[j]], rows_v.at[slot], gsems[slot]
            )

        def wait_gather(j, slot):
            pltpu.make_async_copy(
                table_hbm.at[idx_v.at[j]], rows_v.at[slot], gsems[slot]
            ).wait()

        def store(j, slot):
            pltpu.async_copy(rows_v.at[slot], out_hbm.at[base + j], ssems[slot])

        def wait_store(j, slot):
            pltpu.make_async_copy(
                rows_v.at[slot], out_hbm.at[base + j], ssems[slot]
            ).wait()

        gather(0, 0)

        def body(jj, _):
            for b in range(NBUF):
                j = jj * NBUF + b
                ns = (b + 1) % NBUF

                @pl.when(j >= 1)
                def _():
                    wait_store(j - 1, ns)

                @pl.when(j + 1 < ROWS_PER_W)
                def _():
                    gather(j + 1, ns)

                wait_gather(j, b)
                store(j, b)
            return 0

        lax.fori_loop(0, ROWS_PER_W // NBUF, body, 0, unroll=False)
        wait_store(ROWS_PER_W - 1, (ROWS_PER_W - 1) % NBUF)

    return k


_gather_fn = _make_kernel()


def kernel(token_ids, table):
    return _gather_fn(token_ids.astype(jnp.int32), table)
